# baseline (speedup 1.0000x reference)
import functools

import jax
import jax.numpy as jnp
from jax import lax
from jax.experimental import pallas as pl
from jax.experimental.pallas import tpu as pltpu
from jax.experimental.pallas import tpu_sc as plsc

B = 16384
D = 64
NC = 2
NS = 16
NW = NC * NS
CHUNK = 128
BPW = B // NW
CPW = BPW // CHUNK
GROUPS = BPW // 16

_mesh = plsc.VectorSubcoreMesh(core_axis_name="c", subcore_axis_name="s")


@functools.partial(
    pl.kernel,
    out_type=jax.ShapeDtypeStruct((B,), jnp.float32),
    mesh=_mesh,
    compiler_params=pltpu.CompilerParams(needs_layout_passes=False,
                                         use_tc_tiling_on_sc=False),
    scratch_types=[
        pltpu.VMEM((CPW, CHUNK), jnp.int32),
        pltpu.VMEM((CPW, CHUNK), jnp.int32),
        pltpu.VMEM((BPW, D), jnp.float32),
        pltpu.VMEM((BPW, D), jnp.float32),
        pltpu.VMEM((BPW,), jnp.float32),
        pltpu.VMEM((BPW,), jnp.float32),
        pltpu.VMEM((BPW,), jnp.float32),
        pltpu.VMEM((16,), jnp.float32),
        pltpu.SemaphoreType.DMA,
    ],
)
def _mf_kernel(uid_hbm, iid_hbm, uemb, iemb, ubias, ibias, gbias, out_hbm,
               uidx_v, iidx_v, urows, irows, ub_v, ib_v, out_v, gb_v, sem):
    wid = lax.axis_index("s") * NC + lax.axis_index("c")
    crow0 = wid * CPW

    pltpu.sync_copy(uid_hbm.at[pl.ds(crow0, CPW)], uidx_v)
    pltpu.sync_copy(iid_hbm.at[pl.ds(crow0, CPW)], iidx_v)
    pltpu.sync_copy(gbias, gb_v)

    copies = []
    for j in range(CPW):
        dst = pl.ds(j * CHUNK, CHUNK)
        copies.append(pltpu.make_async_copy(uemb.at[uidx_v.at[j]], urows.at[dst], sem))
        copies.append(pltpu.make_async_copy(iemb.at[iidx_v.at[j]], irows.at[dst], sem))
        copies.append(pltpu.make_async_copy(ubias.at[uidx_v.at[j]], ub_v.at[dst], sem))
        copies.append(pltpu.make_async_copy(ibias.at[iidx_v.at[j]], ib_v.at[dst], sem))
    for c in copies:
        c.start()
    for c in copies:
        c.wait()

    lane = lax.iota(jnp.int32, 16)
    gval = gb_v[...]

    def group_body(gi, _):
        rows = gi * 16 + lane
        acc = ub_v[pl.ds(gi * 16, 16)] + ib_v[pl.ds(gi * 16, 16)]
        for j in range(D):
            jv = jnp.full((16,), j, jnp.int32)
            u = plsc.load_gather(urows, [rows, jv])
            v = plsc.load_gather(irows, [rows, jv])
            acc = acc + u * v
        out_v[pl.ds(gi * 16, 16)] = acc + gval
        return 0

    lax.fori_loop(0, GROUPS, group_body, 0)

    pltpu.sync_copy(out_v, out_hbm.at[pl.ds(wid * BPW, BPW)])


def kernel(user_ids, item_ids, user_emb, item_emb, user_bias, item_bias, global_bias):
    uids = user_ids.astype(jnp.int32).reshape(B // CHUNK, CHUNK)
    iids = item_ids.astype(jnp.int32).reshape(B // CHUNK, CHUNK)
    gb16 = jnp.broadcast_to(global_bias.reshape(()), (16,))
    return _mf_kernel(uids, iids, user_emb, item_emb,
                      user_bias.reshape(-1), item_bias.reshape(-1),
                      gb16)
